# Initial kernel scaffold; baseline (speedup 1.0000x reference)
import jax
import jax.numpy as jnp
from jax.experimental import pallas as pl


def kernel(byte_window, bank):
    raise NotImplementedError("write your pallas kernel here")


# speedup vs baseline: 3.8533x; 3.8533x over previous
import functools

import jax
import jax.numpy as jnp
from jax import lax
from jax.experimental import pallas as pl
from jax.experimental.pallas import tpu as pltpu
from jax.experimental.pallas import tpu_sc as plsc

N_SLOTS = 100000
D_STATE = 8
NGRAM = 16
N_BLADES = 8

NC = 2
NS = 16
L = 16
NW = NC * NS

TOKENS = 16 * 4096
TOK_PER_W = TOKENS // NW
SUB = 512
NSUB = TOK_PER_W // SUB
GROUPS = SUB // L

FNV_INIT = -2128831035
FNV_PRIME = 16777619
TWO32_F = 4294967296.0
INV_SLOTS = 1.0 / float(N_SLOTS)


def _hash_group(bw_v, base, iota):
    byte0 = (base + iota) * NGRAM
    h = jnp.full((L,), FNV_INIT, dtype=jnp.int32)
    for i in range(NGRAM):
        b = plsc.load_gather(bw_v, [byte0 + i])
        h = (h ^ b) * FNV_PRIME
    uf = h.astype(jnp.float32) + jnp.where(h < 0, TWO32_F, 0.0)
    q = (uf * INV_SLOTS).astype(jnp.int32)
    r = h - q * N_SLOTS
    r = jnp.where(r < 0, r + N_SLOTS, r)
    r = jnp.where(r >= N_SLOTS, r - N_SLOTS, r)
    return r


def _body(bw_hbm, bank_hbm, out_hbm, bw_v, idx_v, rows_v, sem):
    wid = lax.axis_index("s") * NC + lax.axis_index("c")
    tok0 = wid * TOK_PER_W
    pltpu.sync_copy(bw_hbm.at[pl.ds(tok0 * NGRAM, TOK_PER_W * NGRAM)], bw_v)
    iota = lax.iota(jnp.int32, L)

    for sub in range(NSUB):
        def group_body(g, _, sub=sub):
            base = sub * SUB + g * L
            addr = _hash_group(bw_v, base, iota)
            row0 = (g * L + iota) * N_BLADES
            for j in range(N_BLADES):
                plsc.store_scatter(idx_v, [row0 + j], addr + j * N_SLOTS)
            return 0

        lax.fori_loop(0, GROUPS, group_body, 0)
        pltpu.async_copy(bank_hbm.at[idx_v], rows_v, sem).wait()
        out_row0 = (tok0 + sub * SUB) * N_BLADES
        pltpu.sync_copy(rows_v, out_hbm.at[pl.ds(out_row0, SUB * N_BLADES), :])


@jax.jit
def _sc_gather(bw_flat, bank2d):
    mesh = plsc.VectorSubcoreMesh(
        core_axis_name="c", subcore_axis_name="s", num_cores=NC, num_subcores=NS
    )
    return pl.kernel(
        _body,
        out_type=jax.ShapeDtypeStruct((TOKENS * N_BLADES, D_STATE), jnp.float32),
        mesh=mesh,
        scratch_types=[
            pltpu.VMEM((TOK_PER_W * NGRAM,), jnp.int32),
            pltpu.VMEM((SUB * N_BLADES,), jnp.int32),
            pltpu.VMEM((SUB * N_BLADES, D_STATE), jnp.float32),
            pltpu.SemaphoreType.DMA,
        ],
        compiler_params=pltpu.CompilerParams(
            needs_layout_passes=False, use_tc_tiling_on_sc=False
        ),
    )(bw_flat, bank2d)


def kernel(byte_window, bank):
    B, S, _ = byte_window.shape
    bw_flat = byte_window.reshape(B * S * NGRAM)
    bank2d = bank.reshape(N_BLADES * N_SLOTS, D_STATE)
    out = _sc_gather(bw_flat, bank2d)
    return out.reshape(B, S, N_BLADES, D_STATE)


# speedup vs baseline: 3.8554x; 1.0005x over previous
import functools

import jax
import jax.numpy as jnp
from jax import lax
from jax.experimental import pallas as pl
from jax.experimental.pallas import tpu as pltpu
from jax.experimental.pallas import tpu_sc as plsc

N_SLOTS = 100000
D_STATE = 8
NGRAM = 16
N_BLADES = 8

NC = 2
NS = 16
L = 16
NW = NC * NS

B_WIN = 16
S_WIN = 4096
TOKENS = B_WIN * S_WIN
TOK_PER_W = TOKENS // NW
SUB = 512
NSUB = TOK_PER_W // SUB
GROUPS = SUB // L
W_PER_B = S_WIN // TOK_PER_W

FNV_INIT = -2128831035
FNV_PRIME = 16777619
TWO32_F = 4294967296.0
INV_SLOTS = 1.0 / float(N_SLOTS)


def _hash_group(bw_v, base, iota):
    tok = base + iota
    h = jnp.full((L,), FNV_INIT, dtype=jnp.int32)
    for i in range(NGRAM):
        b = plsc.load_gather(bw_v, [tok, jnp.full((L,), i, dtype=jnp.int32)])
        h = (h ^ b) * FNV_PRIME
    uf = h.astype(jnp.float32) + jnp.where(h < 0, TWO32_F, 0.0)
    q = (uf * INV_SLOTS).astype(jnp.int32)
    r = h - q * N_SLOTS
    r = jnp.where(r < 0, r + N_SLOTS, r)
    r = jnp.where(r >= N_SLOTS, r - N_SLOTS, r)
    return r


def _body(bw_hbm, bank2, out2, bw_v, idx_v, rows_v, sem):
    wid = lax.axis_index("s") * NC + lax.axis_index("c")
    tok0 = wid * TOK_PER_W
    wb = wid // W_PER_B
    ws = (wid % W_PER_B) * TOK_PER_W
    pltpu.sync_copy(bw_hbm.at[wb, pl.ds(ws, TOK_PER_W), :], bw_v)
    iota = lax.iota(jnp.int32, L)

    for sub in range(NSUB):
        def group_body(g, _, sub=sub):
            base = sub * SUB + g * L
            addr = _hash_group(bw_v, base, iota)
            row0 = (g * L + iota) * N_BLADES
            for j in range(N_BLADES):
                plsc.store_scatter(idx_v, [row0 + j], addr + j * N_SLOTS)
            return 0

        lax.fori_loop(0, GROUPS, group_body, 0)
        pltpu.async_copy(bank2.at[idx_v], rows_v, sem).wait()
        out_row0 = (tok0 + sub * SUB) * N_BLADES
        pltpu.sync_copy(rows_v, out2.at[pl.ds(out_row0, SUB * N_BLADES), :])


@jax.jit
def _sc_gather(byte_window, bank2):
    mesh = plsc.VectorSubcoreMesh(
        core_axis_name="c", subcore_axis_name="s", num_cores=NC, num_subcores=NS
    )
    return pl.kernel(
        _body,
        out_type=jax.ShapeDtypeStruct(
            (TOKENS * N_BLADES, D_STATE), jnp.float32
        ),
        mesh=mesh,
        scratch_types=[
            pltpu.VMEM((TOK_PER_W, NGRAM), jnp.int32),
            pltpu.VMEM((SUB * N_BLADES,), jnp.int32),
            pltpu.VMEM((SUB * N_BLADES, D_STATE), jnp.float32),
            pltpu.SemaphoreType.DMA,
        ],
        compiler_params=pltpu.CompilerParams(
            needs_layout_passes=False, use_tc_tiling_on_sc=False
        ),
    )(byte_window, bank2)


def kernel(byte_window, bank):
    one = (byte_window[0, 0, 0] * 0 + 1).astype(jnp.float32)
    bank2 = bank.reshape(N_BLADES * N_SLOTS, D_STATE) * one
    out = _sc_gather(byte_window, bank2) * one
    return out.reshape(B_WIN, S_WIN, N_BLADES, D_STATE)


# speedup vs baseline: 10.5963x; 2.7485x over previous
import functools

import jax
import jax.numpy as jnp
from jax import lax
from jax.experimental import pallas as pl
from jax.experimental.pallas import tpu as pltpu
from jax.experimental.pallas import tpu_sc as plsc

N_SLOTS = 100000
D_STATE = 8
NGRAM = 16
N_BLADES = 8
ROW = N_BLADES * D_STATE

NC = 2
NS = 16
L = 16
NW = NC * NS

B_WIN = 16
S_WIN = 4096
TOKENS = B_WIN * S_WIN
TOK_PER_W = TOKENS // NW
SUB = 512
NSUB = TOK_PER_W // SUB
GROUPS = SUB // L
W_PER_B = S_WIN // TOK_PER_W

FNV_INIT = -2128831035
FNV_PRIME = 16777619
TWO32_F = 4294967296.0
INV_SLOTS = 1.0 / float(N_SLOTS)


def _hash_group(bw_v, base):
    h = jnp.full((L,), FNV_INIT, dtype=jnp.int32)
    for i in range(NGRAM):
        b = bw_v.at[i][pl.ds(base, L)]
        h = (h ^ b) * FNV_PRIME
    uf = h.astype(jnp.float32) + jnp.where(h < 0, TWO32_F, 0.0)
    q = (uf * INV_SLOTS).astype(jnp.int32)
    r = h - q * N_SLOTS
    r = jnp.where(r < 0, r + N_SLOTS, r)
    r = jnp.where(r >= N_SLOTS, r - N_SLOTS, r)
    return r


def _body(bw_hbm, bank_hbm, out_hbm, bw_v, addr_v, rows_v, stage_v, sem):
    wid = lax.axis_index("s") * NC + lax.axis_index("c")
    wb = wid // W_PER_B
    ws = (wid % W_PER_B) * TOK_PER_W
    pltpu.sync_copy(bw_hbm.at[wb, :, pl.ds(ws, TOK_PER_W)], bw_v)
    iota = lax.iota(jnp.int32, L)

    for sub in range(NSUB):
        def group_body(g, _, sub=sub):
            addr = _hash_group(bw_v, sub * SUB + g * L)
            addr_v[pl.ds(g * L, L)] = addr
            return 0

        lax.fori_loop(0, GROUPS, group_body, 0)
        pltpu.async_copy(bank_hbm.at[addr_v], rows_v, sem).wait()

        def col_body(c, _):
            col = jnp.full((L,), c, dtype=jnp.int32)
            jv = col // D_STATE
            dv = col % D_STATE

            def seg_body(g2, _c):
                rows16 = plsc.load_gather(rows_v, [g2 * L + iota, col])
                plsc.store_scatter(stage_v, [jv, dv, g2 * L + iota], rows16)
                return 0

            lax.fori_loop(0, GROUPS, seg_body, 0)
            return 0

        lax.fori_loop(0, ROW, col_body, 0)
        s0 = ws + sub * SUB
        pltpu.sync_copy(stage_v, out_hbm.at[wb, :, :, pl.ds(s0, SUB)])


@jax.jit
def _sc_gather(bw_t, bank64):
    mesh = plsc.VectorSubcoreMesh(
        core_axis_name="c", subcore_axis_name="s", num_cores=NC, num_subcores=NS
    )
    return pl.kernel(
        _body,
        out_type=jax.ShapeDtypeStruct(
            (B_WIN, N_BLADES, D_STATE, S_WIN), jnp.float32
        ),
        mesh=mesh,
        scratch_types=[
            pltpu.VMEM((NGRAM, TOK_PER_W), jnp.int32),
            pltpu.VMEM((SUB,), jnp.int32),
            pltpu.VMEM((SUB, ROW), jnp.float32),
            pltpu.VMEM((N_BLADES, D_STATE, SUB), jnp.float32),
            pltpu.SemaphoreType.DMA,
        ],
        compiler_params=pltpu.CompilerParams(
            needs_layout_passes=False, use_tc_tiling_on_sc=False
        ),
    )(bw_t, bank64)


def kernel(byte_window, bank):
    bw_t = jnp.transpose(byte_window, (0, 2, 1))
    bank64 = jnp.transpose(bank, (1, 0, 2)).reshape(N_SLOTS, ROW)
    out_t = _sc_gather(bw_t, bank64)
    return jnp.transpose(out_t, (0, 3, 1, 2))
